# baseline (speedup 1.0000x reference)
import functools

import jax
import jax.numpy as jnp
from jax import lax
from jax.experimental import pallas as pl
from jax.experimental.pallas import tpu as pltpu
from jax.experimental.pallas import tpu_sc as plsc

_L = 16
_NC = 2
_NS = 16
_B = 16
_C = 96
_H = 56
_W = 56
_PH = 2
_NCHUNK = _H // _PH
_NBUF = 4
_ROWS = _PH * _W


def _gather(tbl, idx):
    return tbl.at[idx].get(mode="promise_in_bounds")


def _quantize_vec(v, t7, th, th3, th1, cs):
    lo = jnp.where(v >= t7, jnp.int32(8), jnp.int32(0))
    lo = jnp.where(v >= _gather(th3, lo), lo + 4, lo)
    lo = jnp.where(v >= _gather(th1, lo), lo + 2, lo)
    lo = jnp.where(v >= _gather(th, lo), lo + 1, lo)
    return _gather(cs, lo)


def _sc_body(x_hbm, cent_hbm, out0_hbm, out1_hbm, cvmem,
             bi0, bi1, bi2, bi3, bo0, bo1, bo2, bo3,
             ld0, ld1, ld2, ld3, st0, st1, st2, st3):
    cid = lax.axis_index("c")
    sid = lax.axis_index("s")

    pltpu.sync_copy(cent_hbm, cvmem)
    cs, _ = plsc.sort_key_val(cvmem[...], cvmem[...])
    idx16 = lax.broadcasted_iota(jnp.int32, (_L,), 0)
    nxt = jnp.minimum(idx16 + 1, 15)
    th = (cs + _gather(cs, nxt)) * 0.5
    th = jnp.where(idx16 >= 15, jnp.float32(jnp.inf), th)
    th3 = _gather(th, jnp.minimum(idx16 + 3, 15))
    th1 = _gather(th, nxt)
    t7 = _gather(th, jnp.full((_L,), 7, jnp.int32))

    bufs_in = (bi0, bi1, bi2, bi3)
    bufs_out = (bo0, bo1, bo2, bo3)
    ld_sems = (ld0, ld1, ld2, ld3)
    st_sems = (st0, st1, st2, st3)

    def compute_chunk(src, dst):
        def body(i, _):
            for u in range(2):
                r = i * 2 + u
                for start in range(0, _C, _L):
                    v = src[r, pl.ds(start, _L)]
                    dst[r, pl.ds(start, _L)] = _quantize_vec(
                        v, t7, th, th3, th1, cs)
            return 0
        lax.fori_loop(0, _ROWS // 2, body, 0)

    def run_half(share, out_hbm):
        def load(c, b):
            for p in range(_PH):
                src = x_hbm.at[share, sid, c * _PH + p]
                dst = bufs_in[b].at[pl.ds(p * _W, _W)]
                pltpu.async_copy(src, dst, ld_sems[b])

        def store(c, b):
            for p in range(_PH):
                src = bufs_out[b].at[pl.ds(p * _W, _W)]
                dst = out_hbm.at[sid, c * _PH + p]
                pltpu.async_copy(src, dst, st_sems[b])

        def wait_ld(b):
            for p in range(_PH):
                pltpu.make_async_copy(
                    x_hbm.at[share, sid, 0],
                    bufs_in[b].at[pl.ds(0, _W)], ld_sems[b]).wait()

        def wait_st(b):
            for p in range(_PH):
                pltpu.make_async_copy(
                    bufs_out[b].at[pl.ds(0, _W)],
                    out_hbm.at[sid, 0], st_sems[b]).wait()

        def step(c, b, first, last):
            wait_ld(b)
            if not first:
                wait_st(b)
            compute_chunk(bufs_in[b], bufs_out[b])
            store(c, b)
            if not last:
                load(c + _NBUF, b)

        ngroups = _NCHUNK // _NBUF
        for b in range(_NBUF):
            load(b, b)
        for b in range(_NBUF):
            step(b, b, True, False)

        def steady(g, _):
            for b in range(_NBUF):
                step(g * _NBUF + b, b, False, False)
            return 0
        lax.fori_loop(1, ngroups - 1, steady, 0)

        for b in range(_NBUF):
            step((ngroups - 1) * _NBUF + b, b, False, True)
        for b in range(_NBUF):
            wait_st(b)

    pl.when(cid == 0)(lambda: run_half(0, out0_hbm))
    pl.when(cid == 1)(lambda: run_half(1, out1_hbm))


@jax.jit
def _act_share_q(x, centroids):
    mesh = plsc.VectorSubcoreMesh(
        core_axis_name="c", subcore_axis_name="s",
        num_cores=_NC, num_subcores=_NS)
    f = pl.kernel(
        _sc_body,
        out_type=(
            jax.ShapeDtypeStruct((_B, _H, _W, _C), jnp.float32),
            jax.ShapeDtypeStruct((_B, _H, _W, _C), jnp.float32),
        ),
        mesh=mesh,
        scratch_types=(
            [pltpu.VMEM((_L,), jnp.float32)]
            + [pltpu.VMEM((_ROWS, _C), jnp.float32) for _ in range(2 * _NBUF)]
            + [pltpu.SemaphoreType.DMA for _ in range(2 * _NBUF)]
        ),
        compiler_params=pltpu.CompilerParams(
            needs_layout_passes=False, use_tc_tiling_on_sc=True),
    )
    return f(x, centroids)


def kernel(input, centroids):
    xt = jnp.transpose(input, (0, 1, 3, 4, 2))
    o0, o1 = _act_share_q(xt, centroids)
    return (jnp.transpose(o0, (0, 3, 1, 2)), jnp.transpose(o1, (0, 3, 1, 2)))


# speedup vs baseline: 1.0392x; 1.0392x over previous
import jax
import jax.numpy as jnp
from jax import lax
from jax.experimental import pallas as pl
from jax.experimental.pallas import tpu as pltpu
from jax.experimental.pallas import tpu_sc as plsc

_L = 16
_NC = 2
_NS = 16
_B = 16
_C = 96
_H = 56
_W = 56
_PH = 2
_NCHUNK = _H // _PH
_NBUF = 4
_ROWS = _PH * _W


def _gather(tbl, idx):
    return tbl.at[idx].get(mode="promise_in_bounds")


def _quantize_vec(v, t7, th, th3, th1, cs):
    lo = jnp.where(v >= t7, jnp.int32(8), jnp.int32(0))
    lo = jnp.where(v >= _gather(th3, lo), lo + 4, lo)
    lo = jnp.where(v >= _gather(th1, lo), lo + 2, lo)
    lo = jnp.where(v >= _gather(th, lo), lo + 1, lo)
    return _gather(cs, lo)


def _sc_body(x_hbm, cent_hbm, out0_hbm, out1_hbm, cvmem,
             bi0, bi1, bi2, bi3, bo0, bo1, bo2, bo3,
             ld0, ld1, ld2, ld3, st0, st1, st2, st3):
    cid = lax.axis_index("c")
    sid = lax.axis_index("s")

    pltpu.sync_copy(cent_hbm, cvmem)
    cs, _ = plsc.sort_key_val(cvmem[...], cvmem[...])
    idx16 = lax.broadcasted_iota(jnp.int32, (_L,), 0)
    nxt = jnp.minimum(idx16 + 1, 15)
    th = (cs + _gather(cs, nxt)) * 0.5
    th = jnp.where(idx16 >= 15, jnp.float32(jnp.inf), th)
    th3 = _gather(th, jnp.minimum(idx16 + 3, 15))
    th1 = _gather(th, nxt)
    t7 = _gather(th, jnp.full((_L,), 7, jnp.int32))

    bufs_in = (bi0, bi1, bi2, bi3)
    bufs_out = (bo0, bo1, bo2, bo3)
    ld_sems = (ld0, ld1, ld2, ld3)
    st_sems = (st0, st1, st2, st3)

    def compute_chunk(src, dst):
        def body(r, _):
            for start in range(0, _C, _L):
                v = src[r, pl.ds(start, _L)]
                dst[r, pl.ds(start, _L)] = _quantize_vec(
                    v, t7, th, th3, th1, cs)
            return 0
        lax.fori_loop(0, _ROWS, body, 0)

    def run_half(share, out_hbm):
        def load(c, b):
            for p in range(_PH):
                src = x_hbm.at[share, sid, c * _PH + p]
                dst = bufs_in[b].at[pl.ds(p * _W, _W)]
                pltpu.async_copy(src, dst, ld_sems[b])

        def store(c, b):
            for p in range(_PH):
                src = bufs_out[b].at[pl.ds(p * _W, _W)]
                dst = out_hbm.at[sid, c * _PH + p]
                pltpu.async_copy(src, dst, st_sems[b])

        def wait_ld(b):
            for p in range(_PH):
                pltpu.make_async_copy(
                    x_hbm.at[share, sid, 0],
                    bufs_in[b].at[pl.ds(0, _W)], ld_sems[b]).wait()

        def wait_st(b):
            for p in range(_PH):
                pltpu.make_async_copy(
                    bufs_out[b].at[pl.ds(0, _W)],
                    out_hbm.at[sid, 0], st_sems[b]).wait()

        def step(c, b, first, last):
            wait_ld(b)
            if not first:
                wait_st(b)
            compute_chunk(bufs_in[b], bufs_out[b])
            store(c, b)
            if not last:
                load(c + _NBUF, b)

        ngroups = _NCHUNK // _NBUF
        for b in range(_NBUF):
            load(b, b)
        for b in range(_NBUF):
            step(b, b, True, False)

        def steady(g, _):
            for b in range(_NBUF):
                step(g * _NBUF + b, b, False, False)
            return 0
        lax.fori_loop(1, ngroups - 1, steady, 0)

        for b in range(_NBUF):
            step((ngroups - 1) * _NBUF + b, b, False, True)
        for b in range(_NBUF):
            wait_st(b)

    pl.when(cid == 0)(lambda: run_half(0, out0_hbm))
    pl.when(cid == 1)(lambda: run_half(1, out1_hbm))


@jax.jit
def _act_share_q(x, centroids):
    mesh = plsc.VectorSubcoreMesh(
        core_axis_name="c", subcore_axis_name="s",
        num_cores=_NC, num_subcores=_NS)
    f = pl.kernel(
        _sc_body,
        out_type=(
            jax.ShapeDtypeStruct((_B, _H, _W, _C), jnp.float32),
            jax.ShapeDtypeStruct((_B, _H, _W, _C), jnp.float32),
        ),
        mesh=mesh,
        scratch_types=(
            [pltpu.VMEM((_L,), jnp.float32)]
            + [pltpu.VMEM((_ROWS, _C), jnp.float32) for _ in range(2 * _NBUF)]
            + [pltpu.SemaphoreType.DMA for _ in range(2 * _NBUF)]
        ),
        compiler_params=pltpu.CompilerParams(
            needs_layout_passes=False, use_tc_tiling_on_sc=True),
    )
    return f(x, centroids)


def kernel(input, centroids):
    xt = jnp.transpose(input, (0, 1, 3, 4, 2))
    o0, o1 = _act_share_q(xt, centroids)
    return (jnp.transpose(o0, (0, 3, 1, 2)), jnp.transpose(o1, (0, 3, 1, 2)))
